# baseline (speedup 1.0000x reference)
import functools

import jax
import jax.numpy as jnp
from jax import lax
from jax.experimental import pallas as pl
from jax.experimental.pallas import tpu as pltpu
from jax.experimental.pallas import tpu_sc as plsc

_EMB_DIM = 64
_BATCH = 4096
_HIST = 200

_INFO = plsc.get_sparse_core_info()
_NC = _INFO.num_cores
_NS = _INFO.num_subcores
_NW = _NC * _NS
_SLAB = _BATCH // _NW


def _body(idx_hbm, table_hbm, out_hbm, idx_v, i20, i21, rows0, rows1,
          y0, y1, gsem, osem):
    wid = lax.axis_index("s") * _NC + lax.axis_index("c")
    b0 = wid * _SLAB

    iota = jax.lax.iota(jnp.int32, 16)

    def gather_start(l, idx2, rows, sb):
        for k in range(_SLAB // 16):
            idx2[pl.ds(k * 16, 16)] = (
                idx_v[l, pl.ds(k * 16, 16)] >> 1
            )
        pltpu.async_copy(table_hbm.at[idx2], rows, gsem.at[sb])

    def gather_wait(idx2, rows, sb):
        pltpu.make_async_copy(table_hbm.at[idx2], rows, gsem.at[sb]).wait()

    def write_start(l, y, sb):
        for ti in range(8):
            pltpu.async_copy(
                y.at[ti, pl.ds(0, 8), pl.ds(0, 128)],
                out_hbm.at[l, ti, wid],
                osem.at[sb],
            )

    def write_wait(l, y, sb):
        for ti in range(8):
            pltpu.make_async_copy(
                y.at[ti, pl.ds(0, 8), pl.ds(0, 128)],
                out_hbm.at[l, ti, wid],
                osem.at[sb],
            ).wait()

    def transpose(l, rows, y):
        lvec = jnp.full((16,), 0, jnp.int32) + l

        @pl.loop(0, _SLAB, unroll=4)
        def _b(b):
            bvec = jnp.full((16,), 0, jnp.int32) + b
            par = plsc.load_gather(idx_v, [lvec, bvec]) & 1
            off = par * _EMB_DIM
            for ti in range(4):
                dvec = ti * 16 + iota
                v = plsc.load_gather(rows, [bvec, off + ti * 16 + iota])
                plsc.store_scatter(y, [dvec // 8, dvec % 8, bvec], v)

    pltpu.sync_copy(idx_hbm.at[pl.ds(0, _HIST), pl.ds(b0, _SLAB)], idx_v)

    gather_start(0, i20, rows0, 0)
    gather_start(1, i21, rows1, 1)

    @pl.loop(0, _HIST, step=2)
    def _step(g0):
        for sb, idx2, rows, y in ((0, i20, rows0, y0), (1, i21, rows1, y1)):
            l = g0 + sb
            gather_wait(idx2, rows, sb)

            @pl.when(l >= 2)
            def _():
                write_wait(l - 2, y, sb)

            transpose(l, rows, y)
            write_start(l, y, sb)

            @pl.when(l + 2 < _HIST)
            def _():
                gather_start(l + 2, idx2, rows, sb)

    write_wait(_HIST - 2, y0, 0)
    write_wait(_HIST - 1, y1, 1)


_gather = functools.partial(
    pl.kernel,
    mesh=plsc.VectorSubcoreMesh(core_axis_name="c", subcore_axis_name="s"),
    out_type=jax.ShapeDtypeStruct((_HIST, 8, _NW, 8, 128), jnp.float32),
    scratch_types=[
        pltpu.VMEM((_HIST, _SLAB), jnp.int32),
        pltpu.VMEM((_SLAB,), jnp.int32),
        pltpu.VMEM((_SLAB,), jnp.int32),
        pltpu.VMEM((_SLAB, 128), jnp.float32),
        pltpu.VMEM((_SLAB, 128), jnp.float32),
        pltpu.VMEM((8, 8, 129), jnp.float32),
        pltpu.VMEM((8, 8, 129), jnp.float32),
        pltpu.SemaphoreType.DMA((2,)),
        pltpu.SemaphoreType.DMA((2,)),
    ],
    compiler_params=pltpu.CompilerParams(
        use_tc_tiling_on_sc=True, needs_layout_passes=False
    ),
)(_body)


@jax.jit
def kernel(indices, table):
    idx_t = jnp.transpose(indices).astype(jnp.int32)
    table2 = table.reshape(500000, 128)
    out5 = _gather(idx_t, table2)
    y = out5.transpose(0, 1, 3, 2, 4).reshape(_HIST, _EMB_DIM, _BATCH)
    return y.transpose(2, 0, 1)


# speedup vs baseline: 1.8558x; 1.8558x over previous
import functools

import jax
import jax.numpy as jnp
from jax import lax
from jax.experimental import pallas as pl
from jax.experimental.pallas import tpu as pltpu
from jax.experimental.pallas import tpu_sc as plsc

_EMB_DIM = 64
_BATCH = 4096
_HIST = 200

_INFO = plsc.get_sparse_core_info()
_NC = _INFO.num_cores
_NS = _INFO.num_subcores
_NW = _NC * _NS
_SLAB = _BATCH // _NW


def _body(idx_hbm, table_hbm, out_hbm, idx_v, rows0, rows1, y0, y1,
          gsem, osem):
    wid = lax.axis_index("s") * _NC + lax.axis_index("c")
    b0 = wid * _SLAB

    iota = jax.lax.iota(jnp.int32, 16)

    def gather_start(l, rows, sb):
        pltpu.async_copy(table_hbm.at[idx_v.at[l]], rows, gsem.at[sb])

    def gather_wait(l, rows, sb):
        pltpu.make_async_copy(table_hbm.at[idx_v.at[l]], rows,
                              gsem.at[sb]).wait()

    def write_start(l, y, sb):
        for ti in range(8):
            pltpu.async_copy(
                y.at[ti, pl.ds(0, 8), pl.ds(0, 128)],
                out_hbm.at[l, ti, wid],
                osem.at[sb],
            )

    def write_wait(l, y, sb):
        for ti in range(8):
            pltpu.make_async_copy(
                y.at[ti, pl.ds(0, 8), pl.ds(0, 128)],
                out_hbm.at[l, ti, wid],
                osem.at[sb],
            ).wait()

    def transpose(rows, y):
        @pl.loop(0, _SLAB, unroll=8)
        def _b(b):
            bvec = jnp.full((16,), 0, jnp.int32) + b
            for ti in range(4):
                dvec = ti * 16 + iota
                v = rows[b, pl.ds(ti * 16, 16)]
                plsc.store_scatter(y, [dvec // 8, dvec % 8, bvec], v)

    pltpu.sync_copy(idx_hbm.at[pl.ds(0, _HIST), pl.ds(b0, _SLAB)], idx_v)

    gather_start(0, rows0, 0)
    gather_start(1, rows1, 1)

    @pl.loop(0, _HIST, step=2)
    def _step(g0):
        for sb, rows, y in ((0, rows0, y0), (1, rows1, y1)):
            l = g0 + sb
            gather_wait(l, rows, sb)

            @pl.when(l >= 2)
            def _():
                write_wait(l - 2, y, sb)

            transpose(rows, y)
            write_start(l, y, sb)

            @pl.when(l + 2 < _HIST)
            def _():
                gather_start(l + 2, rows, sb)

    write_wait(_HIST - 2, y0, 0)
    write_wait(_HIST - 1, y1, 1)


_gather = functools.partial(
    pl.kernel,
    mesh=plsc.VectorSubcoreMesh(core_axis_name="c", subcore_axis_name="s"),
    out_type=jax.ShapeDtypeStruct((_HIST, 8, _NW, 8, 128), jnp.float32),
    scratch_types=[
        pltpu.VMEM((_HIST, _SLAB), jnp.int32),
        pltpu.VMEM((_SLAB, _EMB_DIM), jnp.float32),
        pltpu.VMEM((_SLAB, _EMB_DIM), jnp.float32),
        pltpu.VMEM((8, 8, 129), jnp.float32),
        pltpu.VMEM((8, 8, 129), jnp.float32),
        pltpu.SemaphoreType.DMA((2,)),
        pltpu.SemaphoreType.DMA((2,)),
    ],
    compiler_params=pltpu.CompilerParams(
        use_tc_tiling_on_sc=False, needs_layout_passes=False
    ),
)(_body)


@jax.jit
def kernel(indices, table):
    idx_t = jnp.transpose(indices).astype(jnp.int32)
    out5 = _gather(idx_t, table)
    y = out5.transpose(0, 1, 3, 2, 4).reshape(_HIST, _EMB_DIM, _BATCH)
    return y.transpose(2, 0, 1)


# speedup vs baseline: 2.5065x; 1.3506x over previous
import functools

import jax
import jax.experimental.layout
import jax.numpy as jnp
from jax import lax
from jax.experimental import pallas as pl
from jax.experimental.pallas import tpu as pltpu
from jax.experimental.pallas import tpu_sc as plsc

_EMB_DIM = 64
_BATCH = 4096
_HIST = 200

_INFO = plsc.get_sparse_core_info()
_NC = _INFO.num_cores
_NS = _INFO.num_subcores
_NW = _NC * _NS
_SLAB = _BATCH // _NW


def _body(idx_hbm, table_hbm, out_hbm, idx_v, rows0, rows1, y0, y1,
          gsem, osem):
    wid = lax.axis_index("s") * _NC + lax.axis_index("c")
    b0 = wid * _SLAB

    iota = jax.lax.iota(jnp.int32, 16)

    def gather_start(l, rows, sb):
        pltpu.async_copy(table_hbm.at[idx_v.at[l]], rows, gsem.at[sb])

    def gather_wait(l, rows, sb):
        pltpu.make_async_copy(table_hbm.at[idx_v.at[l]], rows,
                              gsem.at[sb]).wait()

    def write_start(l, y, sb):
        for ti in range(8):
            pltpu.async_copy(
                y.at[ti, pl.ds(0, 8), pl.ds(0, 128)],
                out_hbm.at[l, ti, wid],
                osem.at[sb],
            )

    def write_wait(l, y, sb):
        for ti in range(8):
            pltpu.make_async_copy(
                y.at[ti, pl.ds(0, 8), pl.ds(0, 128)],
                out_hbm.at[l, ti, wid],
                osem.at[sb],
            ).wait()

    def transpose(rows, y):
        @pl.loop(0, _SLAB, unroll=8)
        def _b(b):
            bvec = jnp.full((16,), 0, jnp.int32) + b
            for ti in range(4):
                dvec = ti * 16 + iota
                v = rows[b, pl.ds(ti * 16, 16)]
                plsc.store_scatter(y, [dvec // 8, dvec % 8, bvec], v)

    pltpu.sync_copy(idx_hbm.at[pl.ds(0, _HIST), pl.ds(b0, _SLAB)], idx_v)

    gather_start(0, rows0, 0)
    gather_start(1, rows1, 1)

    @pl.loop(0, _HIST, step=2)
    def _step(g0):
        for sb, rows, y in ((0, rows0, y0), (1, rows1, y1)):
            l = g0 + sb
            gather_wait(l, rows, sb)

            @pl.when(l >= 2)
            def _():
                write_wait(l - 2, y, sb)

            transpose(rows, y)
            write_start(l, y, sb)

            @pl.when(l + 2 < _HIST)
            def _():
                gather_start(l + 2, rows, sb)

    write_wait(_HIST - 2, y0, 0)
    write_wait(_HIST - 1, y1, 1)


_gather = functools.partial(
    pl.kernel,
    mesh=plsc.VectorSubcoreMesh(core_axis_name="c", subcore_axis_name="s"),
    out_type=jax.ShapeDtypeStruct((_HIST, 8, _NW, 8, 128), jnp.float32),
    scratch_types=[
        pltpu.VMEM((_HIST, _SLAB), jnp.int32),
        pltpu.VMEM((_SLAB, _EMB_DIM), jnp.float32),
        pltpu.VMEM((_SLAB, _EMB_DIM), jnp.float32),
        pltpu.VMEM((8, 8, 129), jnp.float32),
        pltpu.VMEM((8, 8, 129), jnp.float32),
        pltpu.SemaphoreType.DMA((2,)),
        pltpu.SemaphoreType.DMA((2,)),
    ],
    compiler_params=pltpu.CompilerParams(
        use_tc_tiling_on_sc=False, needs_layout_passes=False
    ),
)(_body)


@jax.jit
def kernel(indices, table):
    idx_t = jnp.transpose(indices).astype(jnp.int32)
    tbl = jax.experimental.layout.with_layout_constraint(
        table,
        jax.experimental.layout.Layout(major_to_minor=(0, 1),
                                       tiling=((8, 128), (2, 1))),
    )
    out5 = _gather(idx_t, tbl)
    y = out5.transpose(0, 1, 3, 2, 4).reshape(_HIST, _EMB_DIM, _BATCH)
    return y.transpose(2, 0, 1)
